# Initial kernel scaffold; baseline (speedup 1.0000x reference)
import jax
import jax.numpy as jnp
from jax.experimental import pallas as pl


def kernel(x, W1, b1, W2, b2, W3, b3):
    raise NotImplementedError("write your pallas kernel here")


# speedup vs baseline: 4.7967x; 4.7967x over previous
import jax
import jax.numpy as jnp
import numpy as np
from jax.experimental import pallas as pl
from jax.experimental.pallas import tpu as pltpu

_EPS = 1e-08
_TOPK = 100
_N = 576
_D = 512
_MIN_I32 = np.int32(-(2 ** 31))


def _sortable_keys(f):
    bits = jax.lax.bitcast_convert_type(f, jnp.int32)
    return jnp.where(bits >= 0, bits, jnp.bitwise_xor(jnp.bitwise_not(bits), _MIN_I32))


def _keys_to_float(k):
    bits = jnp.where(k >= 0, k, jnp.bitwise_not(jnp.bitwise_xor(k, _MIN_I32)))
    return jax.lax.bitcast_convert_type(bits, jnp.float32)


def _pgcn_kernel(feat_ref, w1_ref, b1_ref, w2_ref, b2_ref, w3_ref, b3_ref, out_ref):
    feat = feat_ref[0]

    n2 = jnp.sum(feat * feat, axis=1, keepdims=True)
    inv_norm = 1.0 / jnp.maximum(jnp.sqrt(n2), _EPS)
    fn = feat * inv_norm
    sim = jax.lax.dot_general(
        fn, fn, (((1,), (1,)), ((), ())), preferred_element_type=jnp.float32
    )

    keys = _sortable_keys(sim)

    def body(_, carry):
        lo, hi = carry
        mid = (lo >> 1) + (hi >> 1) + (lo & hi & 1)
        count = jnp.sum((keys <= mid).astype(jnp.int32), axis=1, keepdims=True)
        pred = count >= _TOPK
        hi = jnp.where(pred, mid, hi)
        lo = jnp.where(pred, lo, mid + 1)
        return lo, hi

    lo0 = jnp.full((_N, 1), -(2 ** 31), jnp.int32)
    hi0 = jnp.full((_N, 1), 2 ** 31 - 1, jnp.int32)
    lo, _ = jax.lax.fori_loop(0, 32, body, (lo0, hi0))
    thresh = _keys_to_float(lo)

    adj = jnp.where(sim >= thresh, sim, 0.0)

    h = feat
    for w_ref, b_ref in ((w1_ref, b1_ref), (w2_ref, b2_ref), (w3_ref, b3_ref)):
        hw = jax.lax.dot_general(
            h, w_ref[...], (((1,), (0,)), ((), ())), preferred_element_type=jnp.float32
        )
        m = jax.lax.dot_general(
            adj, hw, (((1,), (0,)), ((), ())), preferred_element_type=jnp.float32
        )
        h = jnp.maximum(m + b_ref[...], 0.0)

    out_ref[0] = h


def kernel(x, W1, b1, W2, b2, W3, b3):
    B = x.shape[0]
    feat = x[:, 1:, :]
    b1r = b1.reshape(1, _D)
    b2r = b2.reshape(1, _D)
    b3r = b3.reshape(1, _D)

    wspec = pl.BlockSpec((_D, _D), lambda b: (0, 0))
    bspec = pl.BlockSpec((1, _D), lambda b: (0, 0))

    f_out = pl.pallas_call(
        _pgcn_kernel,
        grid=(B,),
        in_specs=[
            pl.BlockSpec((1, _N, _D), lambda b: (b, 0, 0)),
            wspec, bspec, wspec, bspec, wspec, bspec,
        ],
        out_specs=pl.BlockSpec((1, _N, _D), lambda b: (b, 0, 0)),
        out_shape=jax.ShapeDtypeStruct((B, _N, _D), jnp.float32),
        compiler_params=pltpu.CompilerParams(
            dimension_semantics=("parallel",),
        ),
    )(feat, W1, b1r, W2, b2r, W3, b3r)

    return jnp.concatenate([x[:, :1, :], f_out], axis=1)


# speedup vs baseline: 7.8415x; 1.6348x over previous
import jax
import jax.numpy as jnp
import numpy as np
from jax.experimental import pallas as pl
from jax.experimental.pallas import tpu as pltpu

_EPS = 1e-08
_TOPK = 100
_N = 576
_D = 512
_MIN_I32 = np.int32(-(2 ** 31))
_KEY_NEG_INF = np.int32(-2139095041)
_KEY_POS_INF = np.int32(2139095040)


def _keys_to_float(k):
    bits = jnp.where(k >= 0, k, jnp.bitwise_not(jnp.bitwise_xor(k, _MIN_I32)))
    return jax.lax.bitcast_convert_type(bits, jnp.float32)


def _pgcn_kernel(feat_ref, w1_ref, b1_ref, w2_ref, b2_ref, w3_ref, b3_ref, out_ref):
    feat = feat_ref[0]

    n2 = jnp.sum(feat * feat, axis=1, keepdims=True)
    inv_norm = 1.0 / jnp.maximum(jnp.sqrt(n2), _EPS)
    fn = feat * inv_norm
    sim = jax.lax.dot_general(
        fn, fn, (((1,), (1,)), ((), ())), preferred_element_type=jnp.float32
    )

    def body(_, carry):
        lo, hi = carry
        mid = (lo >> 1) + (hi >> 1) + (lo & hi & 1)
        midf = _keys_to_float(mid)
        count = jnp.sum(
            jnp.where(sim <= midf, 1.0, 0.0), axis=0, keepdims=True
        )
        pred = count >= float(_TOPK)
        hi = jnp.where(pred, mid, hi)
        lo = jnp.where(pred, lo, mid + 1)
        return lo, hi

    lo0 = jnp.full((1, _N), _KEY_NEG_INF, jnp.int32)
    hi0 = jnp.full((1, _N), _KEY_POS_INF, jnp.int32)
    lo, _ = jax.lax.fori_loop(0, 32, body, (lo0, hi0))
    thresh_row = _keys_to_float(lo)
    thresh = thresh_row.reshape(_N, 1)

    adj = jnp.where(sim >= thresh, sim, 0.0)

    h = feat
    for w_ref, b_ref in ((w1_ref, b1_ref), (w2_ref, b2_ref), (w3_ref, b3_ref)):
        hw = jax.lax.dot_general(
            h, w_ref[...], (((1,), (0,)), ((), ())), preferred_element_type=jnp.float32
        )
        m = jax.lax.dot_general(
            adj, hw, (((1,), (0,)), ((), ())), preferred_element_type=jnp.float32
        )
        h = jnp.maximum(m + b_ref[...], 0.0)

    out_ref[0] = h


def kernel(x, W1, b1, W2, b2, W3, b3):
    B = x.shape[0]
    feat = x[:, 1:, :]
    b1r = b1.reshape(1, _D)
    b2r = b2.reshape(1, _D)
    b3r = b3.reshape(1, _D)

    wspec = pl.BlockSpec((_D, _D), lambda b: (0, 0))
    bspec = pl.BlockSpec((1, _D), lambda b: (0, 0))

    f_out = pl.pallas_call(
        _pgcn_kernel,
        grid=(B,),
        in_specs=[
            pl.BlockSpec((1, _N, _D), lambda b: (b, 0, 0)),
            wspec, bspec, wspec, bspec, wspec, bspec,
        ],
        out_specs=pl.BlockSpec((1, _N, _D), lambda b: (b, 0, 0)),
        out_shape=jax.ShapeDtypeStruct((B, _N, _D), jnp.float32),
        compiler_params=pltpu.CompilerParams(
            dimension_semantics=("parallel",),
        ),
    )(feat, W1, b1r, W2, b2r, W3, b3r)

    return jnp.concatenate([x[:, :1, :], f_out], axis=1)


# speedup vs baseline: 7.8791x; 1.0048x over previous
import jax
import jax.numpy as jnp
import numpy as np
from jax.experimental import pallas as pl
from jax.experimental.pallas import tpu as pltpu

_EPS = 1e-08
_TOPK = 100
_N = 576
_D = 512
_MIN_I32 = np.int32(-(2 ** 31))
_KEY_NEG_INF = np.int32(-2139095041)
_KEY_POS_INF = np.int32(2139095040)


def _keys_to_float(k):
    bits = jnp.where(k >= 0, k, jnp.bitwise_not(jnp.bitwise_xor(k, _MIN_I32)))
    return jax.lax.bitcast_convert_type(bits, jnp.float32)


def _pgcn_kernel(feat_ref, w1_ref, b1_ref, w2_ref, b2_ref, w3_ref, b3_ref, out_ref):
    feat = feat_ref[0]

    n2 = jnp.sum(feat * feat, axis=1, keepdims=True)
    inv_norm = 1.0 / jnp.maximum(jnp.sqrt(n2), _EPS)
    fn = feat * inv_norm
    sim = jax.lax.dot_general(
        fn, fn, (((1,), (1,)), ((), ())), preferred_element_type=jnp.float32
    )

    def body(_, carry):
        lo, hi = carry
        mid = (lo >> 1) + (hi >> 1) + (lo & hi & 1)
        midf = _keys_to_float(mid)
        count = jnp.sum(
            jnp.where(sim <= midf, 1.0, 0.0), axis=0, keepdims=True
        )
        pred = count >= float(_TOPK)
        hi = jnp.where(pred, mid, hi)
        lo = jnp.where(pred, lo, mid + 1)
        return lo, hi

    lo0 = jnp.full((1, _N), _KEY_NEG_INF, jnp.int32)
    hi0 = jnp.full((1, _N), _KEY_POS_INF, jnp.int32)
    lo, _ = jax.lax.fori_loop(0, 32, body, (lo0, hi0))
    thresh_row = _keys_to_float(lo)
    thresh = thresh_row.reshape(_N, 1)

    adj = jnp.where(sim >= thresh, sim, 0.0)

    adj_b = adj.astype(jnp.bfloat16)
    h = feat
    for w_ref, b_ref in ((w1_ref, b1_ref), (w2_ref, b2_ref), (w3_ref, b3_ref)):
        hw = jax.lax.dot_general(
            h.astype(jnp.bfloat16),
            w_ref[...].astype(jnp.bfloat16),
            (((1,), (0,)), ((), ())),
            preferred_element_type=jnp.float32,
        )
        m = jax.lax.dot_general(
            adj_b,
            hw.astype(jnp.bfloat16),
            (((1,), (0,)), ((), ())),
            preferred_element_type=jnp.float32,
        )
        h = jnp.maximum(m + b_ref[...], 0.0)

    out_ref[0] = h


def kernel(x, W1, b1, W2, b2, W3, b3):
    B = x.shape[0]
    feat = x[:, 1:, :]
    b1r = b1.reshape(1, _D)
    b2r = b2.reshape(1, _D)
    b3r = b3.reshape(1, _D)

    wspec = pl.BlockSpec((_D, _D), lambda b: (0, 0))
    bspec = pl.BlockSpec((1, _D), lambda b: (0, 0))

    f_out = pl.pallas_call(
        _pgcn_kernel,
        grid=(B,),
        in_specs=[
            pl.BlockSpec((1, _N, _D), lambda b: (b, 0, 0)),
            wspec, bspec, wspec, bspec, wspec, bspec,
        ],
        out_specs=pl.BlockSpec((1, _N, _D), lambda b: (b, 0, 0)),
        out_shape=jax.ShapeDtypeStruct((B, _N, _D), jnp.float32),
        compiler_params=pltpu.CompilerParams(
            dimension_semantics=("parallel",),
        ),
    )(feat, W1, b1r, W2, b2r, W3, b3r)

    return jnp.concatenate([x[:, :1, :], f_out], axis=1)
